# Initial kernel scaffold; baseline (speedup 1.0000x reference)
import jax
import jax.numpy as jnp
from jax.experimental import pallas as pl


def kernel(x, W_enc, b_enc, W_dec, b_dec):
    raise NotImplementedError("write your pallas kernel here")


# speedup vs baseline: 4.3499x; 4.3499x over previous
import jax
import jax.numpy as jnp
from jax.experimental import pallas as pl

K = 32


def _enc_topk_body(x_ref, we_ref, be_ref, sp_ref, *, ht: int):
    j = pl.program_id(1)
    nj = pl.num_programs(1)
    h = jax.lax.dot_general(
        x_ref[...], we_ref[...], (((1,), (1,)), ((), ())),
        preferred_element_type=jnp.float32)
    h = jnp.maximum(h + be_ref[...], 0.0)
    sp_ref[:, pl.ds(j * ht, ht)] = h

    @pl.when(j == nj - 1)
    def _():
        v0 = jnp.max(sp_ref[...], axis=1, keepdims=True)

        def body(_, vprev):
            hh = sp_ref[...]
            return jnp.max(jnp.where(hh < vprev, hh, -1.0), axis=1,
                           keepdims=True)

        vk = jax.lax.fori_loop(0, K - 1, body, v0)
        thr = jnp.maximum(vk, 0.0)
        hh = sp_ref[...]
        sp_ref[...] = jnp.where(hh >= thr, hh, 0.0)


def _decode_body(sp_ref, wd_ref, bd_ref, out_ref):
    j = pl.program_id(1)
    part = jax.lax.dot_general(
        sp_ref[...], wd_ref[...], (((1,), (1,)), ((), ())),
        preferred_element_type=jnp.float32)

    @pl.when(j == 0)
    def _():
        out_ref[...] = part + bd_ref[...]

    @pl.when(j != 0)
    def _():
        out_ref[...] = out_ref[...] + part


def kernel(x, W_enc, b_enc, W_dec, b_dec):
    B, D = x.shape
    H = W_enc.shape[0]

    bt = min(128, B)
    ht = min(1024, H)
    assert B % bt == 0 and H % ht == 0

    import functools
    sparse_hidden = pl.pallas_call(
        functools.partial(_enc_topk_body, ht=ht),
        grid=(B // bt, H // ht),
        in_specs=[
            pl.BlockSpec((bt, D), lambda i, j: (i, 0)),
            pl.BlockSpec((ht, D), lambda i, j: (j, 0)),
            pl.BlockSpec((1, ht), lambda i, j: (0, j)),
        ],
        out_specs=pl.BlockSpec((bt, H), lambda i, j: (i, 0)),
        out_shape=jax.ShapeDtypeStruct((B, H), jnp.float32),
    )(x, W_enc, b_enc.reshape(1, H))

    bt2 = min(512, B)
    ht2 = min(1024, H)
    reconstructed = pl.pallas_call(
        _decode_body,
        grid=(B // bt2, H // ht2),
        in_specs=[
            pl.BlockSpec((bt2, ht2), lambda i, j: (i, j)),
            pl.BlockSpec((D, ht2), lambda i, j: (0, j)),
            pl.BlockSpec((1, D), lambda i, j: (0, 0)),
        ],
        out_specs=pl.BlockSpec((bt2, D), lambda i, j: (i, 0)),
        out_shape=jax.ShapeDtypeStruct((B, D), jnp.float32),
    )(sparse_hidden, W_dec, b_dec.reshape(1, D))

    return (reconstructed, sparse_hidden)


# speedup vs baseline: 5.0326x; 1.1569x over previous
import jax
import jax.numpy as jnp
from jax.experimental import pallas as pl

K = 32


def _enc_topk_body(x_ref, we_ref, be_ref, sp_ref, *, ht: int):
    j = pl.program_id(1)
    nj = pl.num_programs(1)
    h = jax.lax.dot_general(
        x_ref[...], we_ref[...], (((1,), (1,)), ((), ())),
        preferred_element_type=jnp.float32)
    h = jnp.maximum(h + be_ref[...].astype(jnp.float32), 0.0)
    sp_ref[:, pl.ds(j * ht, ht)] = h

    @pl.when(j == nj - 1)
    def _():
        v0 = jnp.max(sp_ref[...], axis=1, keepdims=True)

        def body(_, vprev):
            hh = sp_ref[...]
            return jnp.max(jnp.where(hh < vprev, hh, -1.0), axis=1,
                           keepdims=True)

        vk = jax.lax.fori_loop(0, K - 1, body, v0)
        thr = jnp.maximum(vk, 0.0)
        hh = sp_ref[...]
        sp_ref[...] = jnp.where(hh >= thr, hh, 0.0)


def _decode_body(sp_ref, wd_ref, bd_ref, out_ref):
    j = pl.program_id(1)
    part = jax.lax.dot_general(
        sp_ref[...].astype(jnp.bfloat16), wd_ref[...], (((1,), (1,)), ((), ())),
        preferred_element_type=jnp.float32)

    @pl.when(j == 0)
    def _():
        out_ref[...] = part + bd_ref[...]

    @pl.when(j != 0)
    def _():
        out_ref[...] = out_ref[...] + part


def kernel(x, W_enc, b_enc, W_dec, b_dec):
    B, D = x.shape
    H = W_enc.shape[0]

    bt = min(128, B)
    ht = min(2048, H)
    assert B % bt == 0 and H % ht == 0

    import functools
    sparse_hidden = pl.pallas_call(
        functools.partial(_enc_topk_body, ht=ht),
        grid=(B // bt, H // ht),
        in_specs=[
            pl.BlockSpec((bt, D), lambda i, j: (i, 0)),
            pl.BlockSpec((ht, D), lambda i, j: (j, 0)),
            pl.BlockSpec((1, ht), lambda i, j: (0, j)),
        ],
        out_specs=pl.BlockSpec((bt, H), lambda i, j: (i, 0)),
        out_shape=jax.ShapeDtypeStruct((B, H), jnp.float32),
    )(x.astype(jnp.bfloat16), W_enc.astype(jnp.bfloat16), b_enc.reshape(1, H))

    bt2 = min(512, B)
    ht2 = min(1024, H)
    reconstructed = pl.pallas_call(
        _decode_body,
        grid=(B // bt2, H // ht2),
        in_specs=[
            pl.BlockSpec((bt2, ht2), lambda i, j: (i, j)),
            pl.BlockSpec((D, ht2), lambda i, j: (0, j)),
            pl.BlockSpec((1, D), lambda i, j: (0, 0)),
        ],
        out_specs=pl.BlockSpec((bt2, D), lambda i, j: (i, 0)),
        out_shape=jax.ShapeDtypeStruct((B, D), jnp.float32),
    )(sparse_hidden, W_dec.astype(jnp.bfloat16), b_dec.reshape(1, D))

    return (reconstructed, sparse_hidden)


# speedup vs baseline: 5.1663x; 1.0266x over previous
import jax
import jax.numpy as jnp
from jax.experimental import pallas as pl

K = 32


def _enc_topk_body(x_ref, we_ref, be_ref, sp_ref, *, ht: int):
    j = pl.program_id(1)
    nj = pl.num_programs(1)
    h = jax.lax.dot_general(
        x_ref[...], we_ref[...], (((1,), (1,)), ((), ())),
        preferred_element_type=jnp.float32)
    h = jnp.maximum(h + be_ref[...].astype(jnp.float32), 0.0)
    sp_ref[:, pl.ds(j * ht, ht)] = h

    @pl.when(j == nj - 1)
    def _():
        v0 = jnp.max(sp_ref[...], axis=1, keepdims=True)

        def body(_, vprev):
            hh = sp_ref[...]
            return jnp.max(jnp.where(hh < vprev, hh, -1.0), axis=1,
                           keepdims=True)

        vk = jax.lax.fori_loop(0, K - 1, body, v0)
        thr = jnp.maximum(vk, 0.0)
        hh = sp_ref[...]
        sp_ref[...] = jnp.where(hh >= thr, hh, 0.0)


def _decode_body(sp_ref, wd_ref, bd_ref, out_ref):
    j = pl.program_id(1)
    part = jax.lax.dot_general(
        sp_ref[...].astype(jnp.bfloat16), wd_ref[...], (((1,), (1,)), ((), ())),
        preferred_element_type=jnp.float32)

    @pl.when(j == 0)
    def _():
        out_ref[...] = part + bd_ref[...]

    @pl.when(j != 0)
    def _():
        out_ref[...] = out_ref[...] + part


def kernel(x, W_enc, b_enc, W_dec, b_dec):
    B, D = x.shape
    H = W_enc.shape[0]

    bt = min(128, B)
    ht = min(2048, H)
    assert B % bt == 0 and H % ht == 0

    import functools
    sparse_hidden = pl.pallas_call(
        functools.partial(_enc_topk_body, ht=ht),
        grid=(B // bt, H // ht),
        in_specs=[
            pl.BlockSpec((bt, D), lambda i, j: (i, 0)),
            pl.BlockSpec((ht, D), lambda i, j: (j, 0)),
            pl.BlockSpec((1, ht), lambda i, j: (0, j)),
        ],
        out_specs=pl.BlockSpec((bt, H), lambda i, j: (i, 0)),
        out_shape=jax.ShapeDtypeStruct((B, H), jnp.float32),
    )(x.astype(jnp.bfloat16), W_enc.astype(jnp.bfloat16), b_enc.reshape(1, H))

    bt2 = min(512, B)
    ht2 = min(2048, H)
    reconstructed = pl.pallas_call(
        _decode_body,
        grid=(B // bt2, H // ht2),
        in_specs=[
            pl.BlockSpec((bt2, ht2), lambda i, j: (i, j)),
            pl.BlockSpec((D, ht2), lambda i, j: (0, j)),
            pl.BlockSpec((1, D), lambda i, j: (0, 0)),
        ],
        out_specs=pl.BlockSpec((bt2, D), lambda i, j: (i, 0)),
        out_shape=jax.ShapeDtypeStruct((B, D), jnp.float32),
    )(sparse_hidden, W_dec.astype(jnp.bfloat16), b_dec.reshape(1, D))

    return (reconstructed, sparse_hidden)
